# Initial kernel scaffold; baseline (speedup 1.0000x reference)
import jax
import jax.numpy as jnp
from jax.experimental import pallas as pl


def kernel(x, edge_index, W1, b1, W2, b2):
    raise NotImplementedError("write your pallas kernel here")


# speedup vs baseline: 21.8870x; 21.8870x over previous
import functools

import jax
import jax.numpy as jnp
from jax import lax
from jax.experimental import pallas as pl
from jax.experimental.pallas import tpu as pltpu
from jax.experimental.pallas import tpu_sc as plsc

N_NODES = 10000
IN_CH = 128
HID_CH = 64
N_EDGES = 320000

NC = 2
NS = 16
NW = NC * NS
EPW = N_EDGES // NW
CH = 80
NCH = EPW // CH
NPAD = 10240
RPT = NPAD // NS

_f32 = jnp.float32


def _sc_mesh():
    return plsc.VectorSubcoreMesh(core_axis_name="c", subcore_axis_name="s")


_SC_PARAMS = pltpu.CompilerParams(use_tc_tiling_on_sc=False)


@functools.partial(
    pl.kernel,
    out_type=(jax.ShapeDtypeStruct((N_NODES,), _f32),
              jax.ShapeDtypeStruct((N_NODES,), _f32)),
    mesh=_sc_mesh(),
    scratch_types=[
        pltpu.VMEM((NCH, CH), jnp.int32),
        pltpu.VMEM((CH,), _f32),
        pltpu.VMEM_SHARED((N_NODES,), _f32),
    ],
    compiler_params=_SC_PARAMS,
)
def _sc_degree(ei_hbm, zdeg_hbm, dega, degb, dst_v, ones_v, deg_sh):
    cid = lax.axis_index("c")
    sid = lax.axis_index("s")
    wid = sid * NC + cid

    @pl.when(sid == 0)
    def _():
        pltpu.sync_copy(zdeg_hbm, deg_sh)

    for j in range(CH // 16):
        ones_v[pl.ds(j * 16, 16)] = jnp.ones((16,), _f32)
    pltpu.sync_copy(ei_hbm.at[1, wid], dst_v)
    plsc.subcore_barrier()

    def body(c, carry):
        pltpu.sync_copy(ones_v, deg_sh.at[dst_v.at[c]], add=True)
        return carry

    lax.fori_loop(0, NCH, body, 0)
    plsc.subcore_barrier()

    @pl.when(jnp.logical_and(sid == 0, cid == 0))
    def _():
        pltpu.sync_copy(deg_sh, dega)

    @pl.when(jnp.logical_and(sid == 0, cid == 1))
    def _():
        pltpu.sync_copy(deg_sh, degb)


def _make_sc_agg(F):
    @functools.partial(
        pl.kernel,
        out_type=jax.ShapeDtypeStruct((NC, NPAD, F), _f32),
        mesh=_sc_mesh(),
        scratch_types=[
            pltpu.VMEM((NCH, CH), jnp.int32),
            pltpu.VMEM((NCH, CH), jnp.int32),
            pltpu.VMEM((CH, F), _f32),
            pltpu.VMEM_SHARED((NPAD, F), _f32),
            pltpu.SemaphoreType.DMA,
        ],
        compiler_params=_SC_PARAMS,
    )
    def agg(h_hbm, ei_hbm, z_hbm, outp, src_v, dst_v, rows_v, acc, sem):
        cid = lax.axis_index("c")
        sid = lax.axis_index("s")
        wid = sid * NC + cid

        pltpu.sync_copy(z_hbm.at[pl.ds(sid * RPT, RPT)],
                        acc.at[pl.ds(sid * RPT, RPT)])
        pltpu.sync_copy(ei_hbm.at[0, wid], src_v)
        pltpu.sync_copy(ei_hbm.at[1, wid], dst_v)
        plsc.subcore_barrier()

        def body(c, carry):
            pltpu.async_copy(h_hbm.at[src_v.at[c]], rows_v, sem).wait()
            pltpu.sync_copy(rows_v, acc.at[dst_v.at[c]], add=True)
            return carry

        lax.fori_loop(0, NCH, body, 0)
        plsc.subcore_barrier()

        @pl.when(cid == 0)
        def _():
            pltpu.sync_copy(acc.at[pl.ds(sid * RPT, RPT)],
                            outp.at[0, pl.ds(sid * RPT, RPT)])

        @pl.when(cid == 1)
        def _():
            pltpu.sync_copy(acc.at[pl.ds(sid * RPT, RPT)],
                            outp.at[1, pl.ds(sid * RPT, RPT)])

    return agg


_sc_agg64 = _make_sc_agg(HID_CH)
_sc_agg128 = _make_sc_agg(IN_CH)


_RB = 1000
_GRID = N_NODES // _RB


def _dcol(dega_ref, degb_ref):
    return lax.rsqrt(dega_ref[...] + degb_ref[...] + 1.0)


def _tc_mm1_body(x_ref, w_ref, da_ref, db_ref, o_ref):
    d = _dcol(da_ref, db_ref)
    o_ref[...] = d * jnp.dot(x_ref[...], w_ref[...],
                             preferred_element_type=_f32)


def _tc_mm1(x, W1, da, db):
    return pl.pallas_call(
        _tc_mm1_body,
        grid=(_GRID,),
        in_specs=[
            pl.BlockSpec((_RB, IN_CH), lambda i: (i, 0)),
            pl.BlockSpec((IN_CH, HID_CH), lambda i: (0, 0)),
            pl.BlockSpec((_RB, 1), lambda i: (i, 0)),
            pl.BlockSpec((_RB, 1), lambda i: (i, 0)),
        ],
        out_specs=pl.BlockSpec((_RB, HID_CH), lambda i: (i, 0)),
        out_shape=jax.ShapeDtypeStruct((N_NODES, HID_CH), _f32),
    )(x, W1, da, db)


def _tc_mid_body(aggp_ref, h1p_ref, b1_ref, w2_ref, da_ref, db_ref, o_ref):
    d = _dcol(da_ref, db_ref)
    pre = aggp_ref[0] + aggp_ref[1] + h1p_ref[...]
    z = jnp.maximum(d * pre + b1_ref[...], 0.0)
    o_ref[...] = d * jnp.dot(z, w2_ref[...], preferred_element_type=_f32)


def _tc_mid(aggp, h1p, b1, W2, da, db):
    return pl.pallas_call(
        _tc_mid_body,
        grid=(_GRID,),
        in_specs=[
            pl.BlockSpec((NC, _RB, HID_CH), lambda i: (0, i, 0)),
            pl.BlockSpec((_RB, HID_CH), lambda i: (i, 0)),
            pl.BlockSpec((1, HID_CH), lambda i: (0, 0)),
            pl.BlockSpec((HID_CH, IN_CH), lambda i: (0, 0)),
            pl.BlockSpec((_RB, 1), lambda i: (i, 0)),
            pl.BlockSpec((_RB, 1), lambda i: (i, 0)),
        ],
        out_specs=pl.BlockSpec((_RB, IN_CH), lambda i: (i, 0)),
        out_shape=jax.ShapeDtypeStruct((N_NODES, IN_CH), _f32),
    )(aggp, h1p, b1, W2, da, db)


def _tc_out_body(aggp_ref, h2p_ref, b2_ref, da_ref, db_ref, o_ref):
    d = _dcol(da_ref, db_ref)
    pre = aggp_ref[0] + aggp_ref[1] + h2p_ref[...]
    o_ref[...] = jnp.tanh(d * pre + b2_ref[...])


def _tc_out(aggp, h2p, b2, da, db):
    return pl.pallas_call(
        _tc_out_body,
        grid=(_GRID,),
        in_specs=[
            pl.BlockSpec((NC, _RB, IN_CH), lambda i: (0, i, 0)),
            pl.BlockSpec((_RB, IN_CH), lambda i: (i, 0)),
            pl.BlockSpec((1, IN_CH), lambda i: (0, 0)),
            pl.BlockSpec((_RB, 1), lambda i: (i, 0)),
            pl.BlockSpec((_RB, 1), lambda i: (i, 0)),
        ],
        out_specs=pl.BlockSpec((_RB, IN_CH), lambda i: (i, 0)),
        out_shape=jax.ShapeDtypeStruct((N_NODES, IN_CH), _f32),
    )(aggp, h2p, b2, da, db)


def kernel(x, edge_index, W1, b1, W2, b2):
    ei = edge_index.astype(jnp.int32).reshape(2, NW, NCH, CH)
    zdeg = jnp.zeros((N_NODES,), _f32)
    z64 = jnp.zeros((NPAD, HID_CH), _f32)
    z128 = jnp.zeros((NPAD, IN_CH), _f32)
    b1r = b1.reshape(1, HID_CH)
    b2r = b2.reshape(1, IN_CH)

    dega, degb = _sc_degree(ei, zdeg)
    da = dega.reshape(N_NODES, 1)
    db = degb.reshape(N_NODES, 1)
    h1p = _tc_mm1(x, W1, da, db)
    agg1p = _sc_agg64(h1p, ei, z64)
    h2p = _tc_mid(agg1p, h1p, b1r, W2, da, db)
    agg2p = _sc_agg128(h2p, ei, z128)
    return _tc_out(agg2p, h2p, b2r, da, db)
